# Initial kernel scaffold; baseline (speedup 1.0000x reference)
import jax
import jax.numpy as jnp
from jax.experimental import pallas as pl


def kernel(x, edge_index, adj_values, W):
    raise NotImplementedError("write your pallas kernel here")


# speedup vs baseline: 4.4879x; 4.4879x over previous
import functools

import jax
import jax.numpy as jnp
from jax import lax
from jax.experimental import pallas as pl
from jax.experimental.pallas import tpu as pltpu
from jax.experimental.pallas import tpu_sc as plsc

N = 10000
E = 320000
D = 128

NC = 2
NS = 16
NW = NC * NS

EDGES_PER_W = E // NW
CHUNK = 80
NCHUNK = EDGES_PER_W // CHUNK
ROWS_PER_TILE = 632


def _spmm_body(x_hbm, cols_hbm, rows_hbm, adj_hbm, zeros_hbm, out_hbm,
               cols_v, rows_v, adj_v, msgs_v, agg_sh, sem):
  c = lax.axis_index("c")
  s = lax.axis_index("s")
  wid = s * NC + c

  r0 = jnp.minimum(s * ROWS_PER_TILE, N - ROWS_PER_TILE)
  pltpu.sync_copy(zeros_hbm.at[pl.ds(r0, ROWS_PER_TILE)],
                  agg_sh.at[pl.ds(r0, ROWS_PER_TILE)])
  plsc.subcore_barrier()

  base = wid * EDGES_PER_W

  def chunk_body(k, carry):
    off = base + k * CHUNK
    pltpu.sync_copy(cols_hbm.at[pl.ds(off, CHUNK)], cols_v)
    pltpu.sync_copy(rows_hbm.at[pl.ds(off, CHUNK)], rows_v)
    pltpu.sync_copy(adj_hbm.at[pl.ds(off, CHUNK)], adj_v)
    pltpu.async_copy(x_hbm.at[cols_v], msgs_v, sem).wait()

    def scale_body(g, carry2):
      a16 = adj_v[pl.ds(16 * g, 16)]
      for e in range(16):
        row = 16 * g + e
        ae = a16[e]
        for j in range(D // 16):
          sl = pl.ds(16 * j, 16)
          msgs_v[row, sl] = msgs_v[row, sl] * ae
      return carry2

    lax.fori_loop(0, CHUNK // 16, scale_body, 0)

    pltpu.sync_copy(msgs_v, agg_sh.at[rows_v], add=True)
    return carry

  lax.fori_loop(0, NCHUNK, chunk_body, 0)

  plsc.subcore_barrier()
  pltpu.sync_copy(agg_sh.at[pl.ds(r0, ROWS_PER_TILE)],
                  out_hbm.at[c, pl.ds(r0, ROWS_PER_TILE)])


_spmm = functools.partial(
    pl.kernel,
    out_type=jax.ShapeDtypeStruct((NC, N, D), jnp.float32),
    mesh=plsc.VectorSubcoreMesh(core_axis_name="c", subcore_axis_name="s"),
    scratch_types=[
        pltpu.VMEM((CHUNK,), jnp.int32),
        pltpu.VMEM((CHUNK,), jnp.int32),
        pltpu.VMEM((CHUNK,), jnp.float32),
        pltpu.VMEM((CHUNK, D), jnp.float32),
        pltpu.VMEM_SHARED((N, D), jnp.float32),
        pltpu.SemaphoreType.DMA,
    ],
)(_spmm_body)


def _matmul_relu_body(agg_ref, w_ref, o_ref):
  a = agg_ref[0] + agg_ref[1]
  o_ref[...] = jnp.maximum(
      jnp.dot(a, w_ref[...], preferred_element_type=jnp.float32), 0.0)


BM = 1000


def _matmul_relu(agg, w):
  return pl.pallas_call(
      _matmul_relu_body,
      grid=(N // BM,),
      in_specs=[
          pl.BlockSpec((NC, BM, D), lambda i: (0, i, 0)),
          pl.BlockSpec((D, D), lambda i: (0, 0)),
      ],
      out_specs=pl.BlockSpec((BM, D), lambda i: (i, 0)),
      out_shape=jax.ShapeDtypeStruct((N, D), jnp.float32),
  )(agg, w)


@jax.jit
def kernel(x, edge_index, adj_values, W):
  rows = edge_index[0]
  cols = edge_index[1]
  zeros = jnp.zeros((N, D), jnp.float32)
  agg = _spmm(x, cols, rows, adj_values, zeros)
  return _matmul_relu(agg, W)


# speedup vs baseline: 12.1719x; 2.7121x over previous
import functools

import jax
import jax.numpy as jnp
from jax import lax
from jax.experimental import pallas as pl
from jax.experimental.pallas import tpu as pltpu
from jax.experimental.pallas import tpu_sc as plsc

N = 10000
E = 320000
D = 128

NC = 2
NS = 16
NW = NC * NS

EDGES_PER_W = E // NW
CHUNK = 80
NCHUNK = EDGES_PER_W // CHUNK
NBUF = 4

ROWS_PER_TILE = 632


def _spmm_body(x_hbm, cols_hbm, rows_hbm, adj_hbm, zeros_hbm, out_hbm,
               cols_v, rows_v, adj_v, msgs, agg_sh, *sems):
  gsem = sems[0:NBUF]
  ssem = sems[NBUF:2 * NBUF]
  esem = sems[2 * NBUF:3 * NBUF]
  c = lax.axis_index("c")
  s = lax.axis_index("s")
  wid = s * NC + c
  base = wid * EDGES_PER_W

  r0 = jnp.minimum(s * ROWS_PER_TILE, N - ROWS_PER_TILE)
  pltpu.sync_copy(zeros_hbm.at[pl.ds(r0, ROWS_PER_TILE)],
                  agg_sh.at[pl.ds(r0, ROWS_PER_TILE)])
  plsc.subcore_barrier()

  def edge_copies(k, b):
    off = base + k * CHUNK
    return [
        pltpu.make_async_copy(cols_hbm.at[pl.ds(off, CHUNK)], cols_v.at[b],
                              esem[b]),
        pltpu.make_async_copy(rows_hbm.at[pl.ds(off, CHUNK)], rows_v.at[b],
                              esem[b]),
        pltpu.make_async_copy(adj_hbm.at[pl.ds(off, CHUNK)], adj_v.at[b],
                              esem[b]),
    ]

  def gather_desc(b):
    return pltpu.make_async_copy(x_hbm.at[cols_v.at[b]], msgs.at[b], gsem[b])

  def scatter_desc(b):
    return pltpu.make_async_copy(msgs.at[b], agg_sh.at[rows_v.at[b]], ssem[b])

  def scale_chunk(b):
    def grp_body(grp, carry):
      a16 = adj_v[b, pl.ds(16 * grp, 16)]
      for e in range(16):
        ae = a16[e]
        for j in range(D // 16):
          sl = pl.ds(16 * j, 16)
          msgs[b, 16 * grp + e, sl] = msgs[b, 16 * grp + e, sl] * ae
      return carry

    lax.fori_loop(0, CHUNK // 16, grp_body, 0)

  for b in range(NBUF - 1):
    for d in edge_copies(b, b):
      d.start()
  for b in range(NBUF - 2):
    for d in edge_copies(b, b):
      d.wait()
    gather_desc(b).start()

  def step(k, b):
    gather_desc(b).wait()
    scale_chunk(b)
    pltpu.async_copy(msgs.at[b], agg_sh.at[rows_v.at[b]], ssem[b], add=True)

    b3 = (b + 3) % NBUF
    b2 = (b + 2) % NBUF

    @pl.when(jnp.logical_and(k >= 1, k + 3 < NCHUNK))
    def _wait_prev_scatter():
      scatter_desc(b3).wait()

    @pl.when(k + 3 < NCHUNK)
    def _prefetch_edges():
      for d in edge_copies(k + 3, b3):
        d.start()

    @pl.when(k + 2 < NCHUNK)
    def _issue_gather():
      for d in edge_copies(k + 2, b2):
        d.wait()
      gather_desc(b2).start()

  def chunk_quad(g, carry):
    for b in range(NBUF):
      step(g * NBUF + b, b)
    return carry

  lax.fori_loop(0, (NCHUNK - 1) // NBUF, chunk_quad, 0)

  step(jnp.int32(NCHUNK - 1), (NCHUNK - 1) % NBUF)
  for k in range(NCHUNK - NBUF, NCHUNK):
    scatter_desc(k % NBUF).wait()

  plsc.subcore_barrier()
  pltpu.sync_copy(agg_sh.at[pl.ds(r0, ROWS_PER_TILE)],
                  out_hbm.at[c, pl.ds(r0, ROWS_PER_TILE)])


_spmm = functools.partial(
    pl.kernel,
    out_type=jax.ShapeDtypeStruct((NC, N, D), jnp.float32),
    mesh=plsc.VectorSubcoreMesh(core_axis_name="c", subcore_axis_name="s"),
    scratch_types=[
        pltpu.VMEM((NBUF, CHUNK), jnp.int32),
        pltpu.VMEM((NBUF, CHUNK), jnp.int32),
        pltpu.VMEM((NBUF, CHUNK), jnp.float32),
        pltpu.VMEM((NBUF, CHUNK, D), jnp.float32),
        pltpu.VMEM_SHARED((N, D), jnp.float32),
    ] + [pltpu.SemaphoreType.DMA] * (3 * NBUF),
)(_spmm_body)


def _matmul_relu_body(agg_ref, w_ref, o_ref):
  a = agg_ref[0] + agg_ref[1]
  o_ref[...] = jnp.maximum(
      jnp.dot(a, w_ref[...], preferred_element_type=jnp.float32), 0.0)


BM = 1000


def _matmul_relu(agg, w):
  return pl.pallas_call(
      _matmul_relu_body,
      grid=(N // BM,),
      in_specs=[
          pl.BlockSpec((NC, BM, D), lambda i: (0, i, 0)),
          pl.BlockSpec((D, D), lambda i: (0, 0)),
      ],
      out_specs=pl.BlockSpec((BM, D), lambda i: (i, 0)),
      out_shape=jax.ShapeDtypeStruct((N, D), jnp.float32),
  )(agg, w)


@jax.jit
def kernel(x, edge_index, adj_values, W):
  rows = edge_index[0]
  cols = edge_index[1]
  zeros = jnp.zeros((N, D), jnp.float32)
  agg = _spmm(x, cols, rows, adj_values, zeros)
  return _matmul_relu(agg, W)


# speedup vs baseline: 12.7394x; 1.0466x over previous
import functools

import jax
import jax.numpy as jnp
from jax import lax
from jax.experimental import pallas as pl
from jax.experimental.pallas import tpu as pltpu
from jax.experimental.pallas import tpu_sc as plsc

N = 10000
E = 320000
D = 128

NC = 2
NS = 16
NW = NC * NS

EDGES_PER_W = E // NW
CHUNK = 80
NCHUNK = EDGES_PER_W // CHUNK
NBUF = 4

ROWS_PER_TILE = 632


def _spmm_body(x_hbm, cols_hbm, rows_hbm, adj_hbm, out_hbm,
               cols_v, rows_v, adj_v, msgs, agg_sh, *sems):
  gsem = sems[0:NBUF]
  ssem = sems[NBUF:2 * NBUF]
  esem = sems[2 * NBUF:3 * NBUF]
  zsem = sems[3 * NBUF]
  c = lax.axis_index("c")
  s = lax.axis_index("s")
  wid = s * NC + c
  base = wid * EDGES_PER_W
  r0 = jnp.minimum(s * ROWS_PER_TILE, N - ROWS_PER_TILE)

  def edge_copies(k, b):
    off = base + k * CHUNK
    return [
        pltpu.make_async_copy(cols_hbm.at[pl.ds(off, CHUNK)], cols_v.at[b],
                              esem[b]),
        pltpu.make_async_copy(rows_hbm.at[pl.ds(off, CHUNK)], rows_v.at[b],
                              esem[b]),
        pltpu.make_async_copy(adj_hbm.at[pl.ds(off, CHUNK)], adj_v.at[b],
                              esem[b]),
    ]

  def gather_desc(b):
    return pltpu.make_async_copy(x_hbm.at[cols_v.at[b]], msgs.at[b], gsem[b])

  def scatter_desc(b):
    return pltpu.make_async_copy(msgs.at[b], agg_sh.at[rows_v.at[b]], ssem[b])

  def scale_chunk(b):
    def grp_body(grp, carry):
      a16 = adj_v[b, pl.ds(16 * grp, 16)]
      for e in range(16):
        ae = a16[e]
        for j in range(D // 16):
          sl = pl.ds(16 * j, 16)
          msgs[b, 16 * grp + e, sl] = msgs[b, 16 * grp + e, sl] * ae
      return carry

    lax.fori_loop(0, CHUNK // 16, grp_body, 0)

  zb = NBUF - 1

  def zero_body(z, carry):
    for j in range(D // 16):
      msgs[zb, z, pl.ds(16 * j, 16)] = jnp.zeros((16,), jnp.float32)
    return carry

  lax.fori_loop(0, CHUNK, zero_body, 0)

  zcopies = []
  for i in range(7):
    zcopies.append(pltpu.make_async_copy(
        msgs.at[zb], agg_sh.at[pl.ds(r0 + i * CHUNK, CHUNK)], zsem))
  zcopies.append(pltpu.make_async_copy(
      msgs.at[zb].at[pl.ds(0, 72)],
      agg_sh.at[pl.ds(r0 + 7 * CHUNK, 72)], zsem))
  for d in zcopies:
    d.start()

  for b in range(NBUF - 1):
    for d in edge_copies(b, b):
      d.start()
  for b in range(NBUF - 2):
    for d in edge_copies(b, b):
      d.wait()
    gather_desc(b).start()

  for d in zcopies:
    d.wait()
  plsc.subcore_barrier()

  def step(k, b):
    gather_desc(b).wait()
    scale_chunk(b)
    pltpu.async_copy(msgs.at[b], agg_sh.at[rows_v.at[b]], ssem[b], add=True)

    b3 = (b + 3) % NBUF
    b2 = (b + 2) % NBUF

    @pl.when(jnp.logical_and(k >= 1, k + 3 < NCHUNK))
    def _wait_prev_scatter():
      scatter_desc(b3).wait()

    @pl.when(k + 3 < NCHUNK)
    def _prefetch_edges():
      for d in edge_copies(k + 3, b3):
        d.start()

    @pl.when(k + 2 < NCHUNK)
    def _issue_gather():
      for d in edge_copies(k + 2, b2):
        d.wait()
      gather_desc(b2).start()

  def chunk_quad(g, carry):
    for b in range(NBUF):
      step(g * NBUF + b, b)
    return carry

  lax.fori_loop(0, (NCHUNK - 1) // NBUF, chunk_quad, 0)

  step(jnp.int32(NCHUNK - 1), (NCHUNK - 1) % NBUF)
  for k in range(NCHUNK - NBUF, NCHUNK):
    scatter_desc(k % NBUF).wait()

  plsc.subcore_barrier()
  pltpu.sync_copy(agg_sh.at[pl.ds(r0, ROWS_PER_TILE)],
                  out_hbm.at[c, pl.ds(r0, ROWS_PER_TILE)])


_spmm = functools.partial(
    pl.kernel,
    out_type=jax.ShapeDtypeStruct((NC, N, D), jnp.float32),
    mesh=plsc.VectorSubcoreMesh(core_axis_name="c", subcore_axis_name="s"),
    scratch_types=[
        pltpu.VMEM((NBUF, CHUNK), jnp.int32),
        pltpu.VMEM((NBUF, CHUNK), jnp.int32),
        pltpu.VMEM((NBUF, CHUNK), jnp.float32),
        pltpu.VMEM((NBUF, CHUNK, D), jnp.float32),
        pltpu.VMEM_SHARED((N, D), jnp.float32),
    ] + [pltpu.SemaphoreType.DMA] * (3 * NBUF + 1),
)(_spmm_body)


def _matmul_relu_body(agg_ref, w_ref, o_ref):
  a = agg_ref[0] + agg_ref[1]
  o_ref[...] = jnp.maximum(
      jnp.dot(a, w_ref[...], preferred_element_type=jnp.float32), 0.0)


BM = 1000


def _matmul_relu(agg, w):
  return pl.pallas_call(
      _matmul_relu_body,
      grid=(N // BM,),
      in_specs=[
          pl.BlockSpec((NC, BM, D), lambda i: (0, i, 0)),
          pl.BlockSpec((D, D), lambda i: (0, 0)),
      ],
      out_specs=pl.BlockSpec((BM, D), lambda i: (i, 0)),
      out_shape=jax.ShapeDtypeStruct((N, D), jnp.float32),
  )(agg, w)


@jax.jit
def kernel(x, edge_index, adj_values, W):
  rows = edge_index[0]
  cols = edge_index[1]
  agg = _spmm(x, cols, rows, adj_values)
  return _matmul_relu(agg, W)
